# baseline (speedup 1.0000x reference)
import jax
import jax.numpy as jnp
from jax.experimental import pallas as pl

B = 32
NR_POINTS = 1024
N = B * NR_POINTS
K = 20
N_FPS = 256
EPS = 1e-5


def _bn(x, g, b):
    m = jnp.mean(x, axis=0)
    v = jnp.var(x, axis=0)
    return (x - m) / jnp.sqrt(v + EPS) * g + b


def _knn(pos):
    pg = pos.reshape(B, NR_POINTS, 3)
    def per(p):
        sq = jnp.sum(p * p, axis=-1)
        d2 = sq[:, None] + sq[None, :] - 2.0 * (p @ p.T)
        d2 = d2 + jnp.eye(NR_POINTS, dtype=p.dtype) * 1e10
        _, idx = jax.lax.top_k(-d2, K)
        return idx
    nbr = jax.vmap(per)(pg)
    nbr = nbr + (jnp.arange(B, dtype=nbr.dtype) * NR_POINTS)[:, None, None]
    return nbr.reshape(N, K)


def _fps(pos):
    pg = pos.reshape(B, NR_POINTS, 3)
    def per(p):
        def body(i, state):
            dmin, idxs = state
            last = idxs[i - 1]
            d = jnp.sum((p - p[last]) ** 2, axis=-1)
            dmin = jnp.minimum(dmin, d)
            nxt = jnp.argmax(dmin).astype(jnp.int32)
            return (dmin, idxs.at[i].set(nxt))
        dmin0 = jnp.full((NR_POINTS,), 1e10, dtype=p.dtype)
        idxs0 = jnp.zeros((N_FPS,), dtype=jnp.int32)
        _, idxs = jax.lax.fori_loop(1, N_FPS, body, (dmin0, idxs0))
        return idxs
    idx = jax.vmap(per)(pg)
    idx = idx + (jnp.arange(B, dtype=jnp.int32) * NR_POINTS)[:, None]
    return idx.reshape(B * N_FPS)


def _head_kernel(y_ref, w_ref, b_ref, o_ref):
    o_ref[...] = y_ref[...] @ w_ref[...] + b_ref[...][None, :]


def kernel(pos, edge_index, batch, W1, b1, g1, bt1, W2, b2, g2, bt2, Wn1, bn1, g3, bt3, Wn4, bn4):
    src = _knn(pos)
    rel = pos[src] - pos[:, None, :]
    x = rel.reshape(N * K, 3)
    h = jnp.maximum(x @ W1 + b1, 0.0)
    h = _bn(h, g1, bt1)
    h = jnp.maximum(h @ W2 + b2, 0.0)
    h = _bn(h, g2, bt2)
    features_dd = jnp.max(h.reshape(N, K, 64), axis=1)
    idx = _fps(pos)
    features_fps = features_dd[idx]
    y1 = features_fps @ Wn1 + bn1
    y1 = y1.reshape(B, N_FPS, 512)
    y1 = jnp.max(y1, axis=1)
    y1 = jnp.maximum(y1, 0.0)
    y1 = _bn(y1, g3, bt3)
    y1 = pl.pallas_call(
        _head_kernel,
        out_shape=jax.ShapeDtypeStruct((B, 40), jnp.float32),
    )(y1, Wn4, bn4)
    return jax.nn.log_softmax(y1, axis=1)


# speedup vs baseline: 10.9271x; 10.9271x over previous
import functools

import jax
import jax.numpy as jnp
from jax import lax
from jax.experimental import pallas as pl
from jax.experimental.pallas import tpu as pltpu
from jax.experimental.pallas import tpu_sc as plsc

B = 32
P = 1024
N = B * P
K = 20
NF = 256
EPS = 1e-5
E = N * K
NC = 2
NS = 16
L = 16

_f32 = jnp.float32
_i32 = jnp.int32


def _topk_body(p_ref, pt_ref, idx_ref):
    p = p_ref[...]
    pt = pt_ref[...]
    sqc = jnp.sum(p * p, axis=1, keepdims=True)
    sqr = jnp.sum(pt * pt, axis=0, keepdims=True)
    d2 = sqc + sqr - 2.0 * jnp.dot(p, pt, preferred_element_type=_f32)
    ri = lax.broadcasted_iota(_i32, (P, P), 0)
    ci = lax.broadcasted_iota(_i32, (P, P), 1)
    d2 = jnp.where(ri == ci, _f32(1e10), d2)
    bigi = _i32(1 << 30)
    cols = []
    for t in range(K):
        m = jnp.min(d2, axis=1, keepdims=True)
        sel = jnp.where(d2 <= m, ci, bigi)
        idxt = jnp.min(sel, axis=1, keepdims=True)
        cols.append(idxt)
        d2 = jnp.where(ci == idxt, _f32(3.0e38), d2)
    idx_ref[0] = jnp.concatenate(cols, axis=1)


def _topk(pos, post):
    return pl.pallas_call(
        _topk_body,
        grid=(B,),
        in_specs=[
            pl.BlockSpec((P, 3), lambda c: (c, 0)),
            pl.BlockSpec((3, P), lambda c: (0, c)),
        ],
        out_specs=pl.BlockSpec((1, P, K), lambda c: (c, 0, 0)),
        out_shape=jax.ShapeDtypeStruct((B, P, K), _i32),
    )(pos, post)


def _screl_body(post_hbm, idx_hbm, rel_hbm, posv, idxv, relv):
    c = lax.axis_index("s") * NC + lax.axis_index("c")
    for cc in range(3):
        pltpu.sync_copy(post_hbm.at[pl.ds(cc * N + c * P, P)],
                        posv.at[pl.ds(cc * P, P)])
    pltpu.sync_copy(idx_hbm.at[pl.ds(c * P * K, P * K)], idxv)
    iota = lax.iota(_i32, L)

    def chunk(j, carry):
        j16 = j * L + iota
        for t in range(K):
            src = plsc.load_gather(idxv, [j16 * K + t])
            for cc in range(3):
                g = plsc.load_gather(posv, [src + cc * P])
                cen = plsc.load_gather(posv, [j16 + cc * P])
                plsc.store_scatter(relv, [(t * 3 + cc) * P + j16], g - cen)
        return carry

    lax.fori_loop(0, P // L, chunk, 0)
    pltpu.sync_copy(relv, rel_hbm.at[pl.ds(c * K * 3 * P, K * 3 * P)])


def _screl(post, idx):
    mesh = plsc.VectorSubcoreMesh(core_axis_name="c", subcore_axis_name="s")
    f = functools.partial(
        pl.kernel,
        out_type=jax.ShapeDtypeStruct((B * K * 3 * P,), _f32),
        mesh=mesh,
        compiler_params=pltpu.CompilerParams(needs_layout_passes=False),
        scratch_types=[
            pltpu.VMEM((3 * P,), _f32),
            pltpu.VMEM((P * K,), _i32),
            pltpu.VMEM((K * 3 * P,), _f32),
        ],
    )(_screl_body)
    return f(post, idx).reshape(B * K, 3, P)


def _stats1_body(rel_ref, w1t_ref, b1_ref, s_ref, q_ref):
    c = pl.program_id(0)

    @pl.when(c == 0)
    def _():
        s_ref[...] = jnp.zeros_like(s_ref)
        q_ref[...] = jnp.zeros_like(q_ref)

    w1t = w1t_ref[...]
    b1 = b1_ref[...]
    s = jnp.zeros((64, 1), _f32)
    q = jnp.zeros((64, 1), _f32)
    for t in range(K):
        x = rel_ref[t]
        h1 = jnp.maximum(jnp.dot(w1t, x, preferred_element_type=_f32) + b1, 0.0)
        s = s + jnp.sum(h1, axis=1, keepdims=True)
        q = q + jnp.sum(h1 * h1, axis=1, keepdims=True)
    s_ref[...] += s
    q_ref[...] += q


def _stats1(rel, w1t, b1c):
    return pl.pallas_call(
        _stats1_body,
        grid=(B,),
        in_specs=[
            pl.BlockSpec((K, 3, P), lambda c: (c, 0, 0)),
            pl.BlockSpec((64, 3), lambda c: (0, 0)),
            pl.BlockSpec((64, 1), lambda c: (0, 0)),
        ],
        out_specs=[
            pl.BlockSpec((64, 1), lambda c: (0, 0)),
            pl.BlockSpec((64, 1), lambda c: (0, 0)),
        ],
        out_shape=[
            jax.ShapeDtypeStruct((64, 1), _f32),
            jax.ShapeDtypeStruct((64, 1), _f32),
        ],
    )(rel, w1t, b1c)


def _conv2_body(rel_ref, w1t_ref, b1_ref, w2ft_ref, b2f_ref,
                mx_ref, mn_ref, s_ref, q_ref):
    c = pl.program_id(0)

    @pl.when(c == 0)
    def _():
        s_ref[...] = jnp.zeros_like(s_ref)
        q_ref[...] = jnp.zeros_like(q_ref)

    w1t = w1t_ref[...]
    b1 = b1_ref[...]
    w2ft = w2ft_ref[...]
    b2f = b2f_ref[...]
    s = jnp.zeros((64, 1), _f32)
    q = jnp.zeros((64, 1), _f32)
    mx = jnp.full((64, P), -3.0e38, _f32)
    mn = jnp.full((64, P), 3.0e38, _f32)
    for t in range(K):
        x = rel_ref[t]
        h1 = jnp.maximum(jnp.dot(w1t, x, preferred_element_type=_f32) + b1, 0.0)
        h2 = jnp.maximum(jnp.dot(w2ft, h1, preferred_element_type=_f32) + b2f, 0.0)
        s = s + jnp.sum(h2, axis=1, keepdims=True)
        q = q + jnp.sum(h2 * h2, axis=1, keepdims=True)
        mx = jnp.maximum(mx, h2)
        mn = jnp.minimum(mn, h2)
    mx_ref[0] = mx
    mn_ref[0] = mn
    s_ref[...] += s
    q_ref[...] += q


def _conv2(rel, w1t, b1c, w2ft, b2f):
    return pl.pallas_call(
        _conv2_body,
        grid=(B,),
        in_specs=[
            pl.BlockSpec((K, 3, P), lambda c: (c, 0, 0)),
            pl.BlockSpec((64, 3), lambda c: (0, 0)),
            pl.BlockSpec((64, 1), lambda c: (0, 0)),
            pl.BlockSpec((64, 64), lambda c: (0, 0)),
            pl.BlockSpec((64, 1), lambda c: (0, 0)),
        ],
        out_specs=[
            pl.BlockSpec((1, 64, P), lambda c: (c, 0, 0)),
            pl.BlockSpec((1, 64, P), lambda c: (c, 0, 0)),
            pl.BlockSpec((64, 1), lambda c: (0, 0)),
            pl.BlockSpec((64, 1), lambda c: (0, 0)),
        ],
        out_shape=[
            jax.ShapeDtypeStruct((B, 64, P), _f32),
            jax.ShapeDtypeStruct((B, 64, P), _f32),
            jax.ShapeDtypeStruct((64, 1), _f32),
            jax.ShapeDtypeStruct((64, 1), _f32),
        ],
    )(rel, w1t, b1c, w2ft, b2f)


def _scfps_body(post_hbm, out_hbm, posv, dminv, idxsv):
    c = lax.axis_index("s") * NC + lax.axis_index("c")
    for cc in range(3):
        pltpu.sync_copy(post_hbm.at[pl.ds(cc * N + c * P, P)],
                        posv.at[pl.ds(cc * P, P)])
    iota = lax.iota(_i32, L)
    zl = jnp.zeros((L,), _i32)
    lane0 = iota == 0
    for j in range(P // L):
        dminv[pl.ds(j * L, L)] = jnp.full((L,), 1e10, _f32)
    plsc.store_scatter(idxsv, [zl], zl, mask=lane0)

    def step(i, last):
        lastv = jnp.full((L,), last, _i32)
        lx = plsc.load_gather(posv, [lastv])
        ly = plsc.load_gather(posv, [lastv + P])
        lz = plsc.load_gather(posv, [lastv + 2 * P])

        def chunk(j, carry):
            bval, bidx = carry
            j16 = j * L + iota
            xx = plsc.load_gather(posv, [j16])
            yy = plsc.load_gather(posv, [j16 + P])
            zz = plsc.load_gather(posv, [j16 + 2 * P])
            dx = xx - lx
            dy = yy - ly
            dz = zz - lz
            d = (dx * dx + dy * dy) + dz * dz
            dm = plsc.load_gather(dminv, [j16])
            dm2 = jnp.minimum(dm, d)
            plsc.store_scatter(dminv, [j16], dm2)
            better = dm2 > bval
            bval = jnp.where(better, dm2, bval)
            bidx = jnp.where(better, j16, bidx)
            return bval, bidx

        bval, bidx = lax.fori_loop(
            0, P // L, chunk,
            (jnp.full((L,), -3.0e38, _f32), jnp.zeros((L,), _i32)))
        m = jnp.max(bval)
        cand = jnp.where(bval == m, bidx, _i32(1 << 30))
        bi = jnp.min(cand)
        plsc.store_scatter(idxsv, [jnp.full((L,), i, _i32)],
                           jnp.full((L,), bi, _i32), mask=lane0)
        return bi

    lax.fori_loop(1, NF, step, _i32(0))
    pltpu.sync_copy(idxsv, out_hbm.at[pl.ds(c * NF, NF)])


def _scfps(post):
    mesh = plsc.VectorSubcoreMesh(core_axis_name="c", subcore_axis_name="s")
    f = functools.partial(
        pl.kernel,
        out_type=jax.ShapeDtypeStruct((B * NF,), _i32),
        mesh=mesh,
        compiler_params=pltpu.CompilerParams(needs_layout_passes=False),
        scratch_types=[
            pltpu.VMEM((3 * P,), _f32),
            pltpu.VMEM((P,), _f32),
            pltpu.VMEM((NF,), _i32),
        ],
    )(_scfps_body)
    return f(post).reshape(B, 1, NF)


def _head_body(mx_ref, mn_ref, idx_ref, a2_ref, c2_ref, wn1_ref, bn1_ref,
               y_ref):
    a2 = a2_ref[...]
    c2 = c2_ref[...]
    f2 = jnp.where(a2 >= 0.0, mx_ref[0], mn_ref[0]) * a2 + c2
    idx = idx_ref[0, 0, :]
    ri = lax.broadcasted_iota(_i32, (P, NF), 0)
    onehot = (ri == idx[None, :]).astype(_f32)
    fsel = jnp.dot(f2, onehot, preferred_element_type=_f32)
    y = lax.dot_general(fsel, wn1_ref[...],
                        (((0,), (0,)), ((), ())),
                        preferred_element_type=_f32)
    y = y + bn1_ref[...]
    y1 = jnp.max(y, axis=0, keepdims=True)
    y_ref[0] = jnp.maximum(y1, 0.0)


def _head(mxall, mnall, idxfps, a2c, c2c, wn1, bn1r):
    return pl.pallas_call(
        _head_body,
        grid=(B,),
        in_specs=[
            pl.BlockSpec((1, 64, P), lambda c: (c, 0, 0)),
            pl.BlockSpec((1, 64, P), lambda c: (c, 0, 0)),
            pl.BlockSpec((1, 1, NF), lambda c: (c, 0, 0)),
            pl.BlockSpec((64, 1), lambda c: (0, 0)),
            pl.BlockSpec((64, 1), lambda c: (0, 0)),
            pl.BlockSpec((64, 512), lambda c: (0, 0)),
            pl.BlockSpec((1, 512), lambda c: (0, 0)),
        ],
        out_specs=pl.BlockSpec((1, 1, 512), lambda c: (c, 0, 0)),
        out_shape=jax.ShapeDtypeStruct((B, 1, 512), _f32),
    )(mxall, mnall, idxfps, a2c, c2c, wn1, bn1r)


def _final_body(y_ref, g3_ref, bt3_ref, wn4_ref, bn4_ref, o_ref):
    y = y_ref[...]
    m = jnp.mean(y, axis=0, keepdims=True)
    v = jnp.mean((y - m) * (y - m), axis=0, keepdims=True)
    yn = (y - m) / jnp.sqrt(v + EPS) * g3_ref[...] + bt3_ref[...]
    z = jnp.dot(yn, wn4_ref[...], preferred_element_type=_f32) + bn4_ref[...]
    zmax = jnp.max(z, axis=1, keepdims=True)
    ze = z - zmax
    lse = jnp.log(jnp.sum(jnp.exp(ze), axis=1, keepdims=True))
    o_ref[...] = ze - lse


def _final(y1, g3r, bt3r, wn4, bn4r):
    return pl.pallas_call(
        _final_body,
        out_shape=jax.ShapeDtypeStruct((B, 40), _f32),
    )(y1, g3r, bt3r, wn4, bn4r)


def kernel(pos, edge_index, batch, W1, b1, g1, bt1, W2, b2, g2, bt2,
           Wn1, bn1, g3, bt3, Wn4, bn4):
    post = pos.T
    postf = post.reshape(3 * N)
    idx = _topk(pos, post)
    rel = _screl(postf, idx.reshape(B * P * K))

    w1t = W1.T
    b1c = b1[:, None]
    s1, q1 = _stats1(rel, w1t, b1c)
    mean1 = s1 / E
    var1 = q1 / E - mean1 * mean1
    a1 = (g1[:, None] / jnp.sqrt(var1 + EPS))
    c1 = bt1[:, None] - mean1 * a1
    w2ft = (W2 * a1).T
    b2f = (c1[:, 0] @ W2 + b2)[:, None]

    mxall, mnall, s2, q2 = _conv2(rel, w1t, b1c, w2ft, b2f)
    mean2 = s2 / E
    var2 = q2 / E - mean2 * mean2
    a2 = g2[:, None] / jnp.sqrt(var2 + EPS)
    c2 = bt2[:, None] - mean2 * a2

    idxfps = _scfps(postf)
    y1 = _head(mxall, mnall, idxfps, a2, c2, Wn1, bn1[None, :])
    y1 = y1.reshape(B, 512)
    return _final(y1, g3[None, :], bt3[None, :], Wn4, bn4[None, :])
